# baseline (speedup 1.0000x reference)
import functools

import jax
import jax.numpy as jnp
from jax import lax
from jax.experimental import pallas as pl
from jax.experimental.pallas import tpu as pltpu
from jax.experimental.pallas import tpu_sc as plsc

_CHUNK = 128
_NBUF = 5


def _emb_lookup(ids_pair, ids_par, table_wide, n_steps, nc, nw):
    N = ids_pair.shape[0]
    n_per_w = n_steps * _CHUNK
    mesh = plsc.VectorSubcoreMesh(core_axis_name="c", subcore_axis_name="s")

    @functools.partial(
        pl.kernel,
        mesh=mesh,
        out_type=jax.ShapeDtypeStruct((N // 2, 128), jnp.float32),
        compiler_params=pltpu.CompilerParams(use_tc_tiling_on_sc=True),
        scratch_types=[
            pltpu.VMEM((n_per_w,), jnp.int32),
            pltpu.VMEM((n_per_w + 16,), jnp.int32),
            pltpu.VMEM((_NBUF, _CHUNK, 128), jnp.float32),
            pltpu.VMEM((2, _CHUNK // 2, 128), jnp.float32),
            pltpu.SemaphoreType.DMA,
            pltpu.SemaphoreType.DMA,
        ],
    )
    def emb(pair_hbm, par_hbm, tab_hbm, out_hbm, pair_v, par_v, rows_v,
            stage_v, gsem, osem):
        wid = lax.axis_index("s") * nc + lax.axis_index("c")
        base = wid * n_per_w
        pltpu.sync_copy(pair_hbm.at[pl.ds(base, n_per_w)], pair_v)
        pltpu.sync_copy(par_hbm.at[pl.ds(base, n_per_w)],
                        par_v.at[pl.ds(0, n_per_w)])

        def gather_copy(ci, buf):
            return pltpu.make_async_copy(
                tab_hbm.at[pair_v.at[pl.ds(ci * _CHUNK, _CHUNK)]],
                rows_v.at[buf],
                gsem,
            )

        def out_copy(ci, sbuf):
            return pltpu.make_async_copy(
                stage_v.at[sbuf],
                out_hbm.at[pl.ds(wid * (n_per_w // 2) + ci * (_CHUNK // 2),
                                 _CHUNK // 2)],
                osem,
            )

        def extract(ci, buf, sbuf):
            rbuf = rows_v.at[buf]
            sb = stage_v.at[sbuf]

            def row_body(r):
                p = par_v[pl.ds(ci * _CHUNK + r, 16)][0]
                off = p * 64
                jj = r >> 1
                cc = (r & 1) * 64
                for q in range(4):
                    sb[jj, pl.ds(cc + q * 16, 16)] = rbuf[
                        r, pl.ds(off + q * 16, 16)
                    ]

            pl.loop(0, _CHUNK)(row_body)

        for b in range(_NBUF):
            gather_copy(b, b).start()

        def body(g):
            for i in range(_NBUF):
                j = g + i
                gather_copy(j, i).wait()
                extract(j, i, i % 2)
                out_copy(j, i % 2).start()
                out_copy(j, i % 2).wait()
                gather_copy(j + _NBUF, i).start()

        pl.loop(0, n_steps - _NBUF, step=_NBUF)(body)

        for i in range(_NBUF):
            j = n_steps - _NBUF + i
            gather_copy(j, i).wait()
            extract(j, i, i % 2)
            out_copy(j, i % 2).start()
            out_copy(j, i % 2).wait()

    return emb(ids_pair, ids_par, table_wide)


def kernel(ids, table):
    B, H = ids.shape
    V, D = table.shape
    N = B * H
    info = plsc.get_sparse_core_info()
    nc, ns = info.num_cores, info.num_subcores
    nw = nc * ns
    n_steps = N // (nw * _CHUNK)
    ids_flat = ids.reshape(N).astype(jnp.int32)
    ids_pair = ids_flat >> 1
    ids_par = ids_flat & 1
    table_wide = table.reshape(V * D // 128, 128)
    out = _emb_lookup(ids_pair, ids_par, table_wide, n_steps, nc, nw)
    return out.reshape(B, H, D)
